# Initial kernel scaffold; baseline (speedup 1.0000x reference)
import jax
import jax.numpy as jnp
from jax.experimental import pallas as pl


def kernel(q, k, attn_mask):
    raise NotImplementedError("write your pallas kernel here")


# speedup vs baseline: 6.9781x; 6.9781x over previous
import jax
import jax.numpy as jnp
from jax.experimental import pallas as pl


def _head_body(q_ref, qT_ref, k_ref, kT_ref, m_ref, o_ref):
    S = q_ref.shape[1]
    D = q_ref.shape[2]
    f32 = jnp.float32

    ii = jax.lax.broadcasted_iota(jnp.int32, (S, S), 0)
    jj = jax.lax.broadcasted_iota(jnp.int32, (S, S), 1)
    j_lt_i = jj < ii
    i_lt_j = ii < jj

    q = q_ref[0]
    qT = qT_ref[0]
    k = k_ref[0]
    kT = kT_ref[0]

    acc = jnp.zeros((S, S), f32)
    for d in range(D):
        qc = q[:, d:d + 1]
        qr = qT[d:d + 1, :]
        kc = k[:, d:d + 1]
        kr = kT[d:d + 1, :]
        q_cnt = (qr < qc) | ((qr == qc) & j_lt_i)
        rank_q = jnp.sum(q_cnt.astype(f32), axis=1, keepdims=True)
        k_cnt = (kc < kr) | ((kc == kr) & i_lt_j)
        rank_k = jnp.sum(k_cnt.astype(f32), axis=0, keepdims=True)
        match = rank_q == rank_k
        diff = qc - kr
        vals = jnp.exp(-(diff * diff))
        acc = acc + jnp.where(match, vals, 0.0)

    masked = jnp.where(m_ref[0] != 0, 0.0, acc * (1.0 / D))
    o_ref[0] = masked


def kernel(q, k, attn_mask):
    B, H, S, D = q.shape
    q3 = q.reshape(H, S, D)
    k3 = k.reshape(H, S, D)
    qT = jnp.transpose(q3, (0, 2, 1))
    kT = jnp.transpose(k3, (0, 2, 1))
    m3 = attn_mask.reshape(H, S, S).astype(jnp.int8)

    out = pl.pallas_call(
        _head_body,
        grid=(H,),
        in_specs=[
            pl.BlockSpec((1, S, D), lambda h: (h, 0, 0)),
            pl.BlockSpec((1, D, S), lambda h: (h, 0, 0)),
            pl.BlockSpec((1, S, D), lambda h: (h, 0, 0)),
            pl.BlockSpec((1, D, S), lambda h: (h, 0, 0)),
            pl.BlockSpec((1, S, S), lambda h: (h, 0, 0)),
        ],
        out_specs=pl.BlockSpec((1, S, S), lambda h: (h, 0, 0)),
        out_shape=jax.ShapeDtypeStruct((H, S, S), jnp.float32),
    )(q3, qT, k3, kT, m3)
    return out.reshape(B, H, S, S)


# speedup vs baseline: 8.1731x; 1.1713x over previous
import functools

import jax
import jax.numpy as jnp
from jax import lax
from jax.experimental import pallas as pl
from jax.experimental.pallas import tpu as pltpu
from jax.experimental.pallas import tpu_sc as plsc

_H, _S, _D = 8, 512, 64
_NSUB = 16
_CPW = _H * _D // _NSUB
_ROWS = 128


def _rank_body(q_ref, qT_ref, k_ref, kT_ref, rq_ref, rk_ref):
    S, D = _S, _D
    ii = lax.broadcasted_iota(jnp.int32, (S, S), 0)
    jj = lax.broadcasted_iota(jnp.int32, (S, S), 1)
    i_lt_j = ii < jj

    q = q_ref[0]
    qT = qT_ref[0]
    k = k_ref[0]
    kT = kT_ref[0]
    for d in range(D):
        qc = q[:, d:d + 1]
        qr = qT[d:d + 1, :]
        kc = k[:, d:d + 1]
        kr = kT[d:d + 1, :]
        mq = (qc < qr) | ((qc == qr) & i_lt_j)
        rq_ref[0, d:d + 1, :] = jnp.sum(mq.astype(jnp.int32), axis=0,
                                        keepdims=True)
        mk = (kc < kr) | ((kc == kr) & i_lt_j)
        rk_ref[0, d:d + 1, :] = jnp.sum(mk.astype(jnp.int32), axis=0,
                                        keepdims=True)


def _mask_body(a_ref, m_ref, o_ref):
    o_ref[0] = jnp.where(m_ref[0] != 0, 0.0, a_ref[0] * (1.0 / _D))


def _sc_body(qT_hbm, kT_hbm, rq_hbm, rk_hbm, out_hbm,
             qv, kv, rqv, rkv, qs, ks, qi_a, ki_a, tmp_col, tmp_w,
             colb, wb, outbuf, stg_col, stg_w):
    f32 = jnp.float32
    i32 = jnp.int32
    wid = lax.axis_index("s")
    iota = lax.iota(i32, 16)

    def per_channel(ch, _):
        c = wid * _CPW + ch
        pltpu.sync_copy(qT_hbm.at[pl.ds(c * _S, _S)], qv)
        pltpu.sync_copy(kT_hbm.at[pl.ds(c * _S, _S)], kv)
        pltpu.sync_copy(rq_hbm.at[pl.ds(c * _S, _S)], rqv)
        pltpu.sync_copy(rk_hbm.at[pl.ds(c * _S, _S)], rkv)

        def pass_a(t, _):
            sl = pl.ds(t * 16, 16)
            iv = iota + t * 16
            plsc.store_scatter(qs, [rqv[sl]], qv[sl])
            plsc.store_scatter(qi_a, [rqv[sl]], iv)
            plsc.store_scatter(ks, [rkv[sl]], kv[sl])
            plsc.store_scatter(ki_a, [rkv[sl]], iv)
            return _
        lax.fori_loop(0, 32, pass_a, None)

        def pass_b(t, _):
            sl = pl.ds(t * 16, 16)
            dqk = qs[sl] - ks[sl]
            w = jnp.exp(-(dqk * dqk))
            qiv = qi_a[sl]
            plsc.store_scatter(tmp_col, [qiv], ki_a[sl])
            plsc.store_scatter(tmp_w, [qiv], w)
            return _
        lax.fori_loop(0, 32, pass_b, None)

        cl = wid * _CPW + ch
        pltpu.sync_copy(tmp_col, stg_col.at[cl])
        pltpu.sync_copy(tmp_w, stg_w.at[cl])
        return _
    lax.fori_loop(0, _CPW, per_channel, None)

    plsc.subcore_barrier()

    head = wid // 2
    zeros16 = jnp.zeros((16,), f32)
    for half in range(2):
        i0 = (wid % 2) * 2 * _ROWS + half * _ROWS

        def load_stage(ch, _):
            r = head * _D + ch
            pltpu.sync_copy(stg_col.at[r, pl.ds(i0, _ROWS)], colb.at[ch])
            pltpu.sync_copy(stg_w.at[r, pl.ds(i0, _ROWS)], wb.at[ch])
            return _
        lax.fori_loop(0, _D, load_stage, None)

        def zero_chunk(t, _):
            outbuf[pl.ds(t * 16, 16)] = zeros16
            return _
        lax.fori_loop(0, _ROWS * _S // 16, zero_chunk, None)

        def per_ch(ch, _):
            for rb in range(_ROWS // 16):
                sl = pl.ds(rb * 16, 16)
                rows_v = iota + rb * 16
                colv = colb[ch, sl]
                wv = wb[ch, sl]
                plsc.addupdate_scatter(outbuf, [rows_v * _S + colv], wv)
            return _
        lax.fori_loop(0, _D, per_ch, None)

        pltpu.sync_copy(outbuf,
                        out_hbm.at[pl.ds((head * _S + i0) * _S,
                                         _ROWS * _S)])


_sc_scatter = functools.partial(
    pl.kernel,
    out_type=jax.ShapeDtypeStruct((_H * _S * _S,), jnp.float32),
    mesh=plsc.VectorSubcoreMesh(core_axis_name="c", subcore_axis_name="s",
                                num_cores=1),
    compiler_params=pltpu.CompilerParams(needs_layout_passes=False),
    scratch_types=[
        pltpu.VMEM((_S,), jnp.float32),
        pltpu.VMEM((_S,), jnp.float32),
        pltpu.VMEM((_S,), jnp.int32),
        pltpu.VMEM((_S,), jnp.int32),
        pltpu.VMEM((_S,), jnp.float32),
        pltpu.VMEM((_S,), jnp.float32),
        pltpu.VMEM((_S,), jnp.int32),
        pltpu.VMEM((_S,), jnp.int32),
        pltpu.VMEM((_S,), jnp.int32),
        pltpu.VMEM((_S,), jnp.float32),
        pltpu.VMEM((_D, _ROWS), jnp.int32),
        pltpu.VMEM((_D, _ROWS), jnp.float32),
        pltpu.VMEM((_ROWS * _S,), jnp.float32),
        pltpu.VMEM_SHARED((_H * _D, _S), jnp.int32),
        pltpu.VMEM_SHARED((_H * _D, _S), jnp.float32),
    ],
)(_sc_body)


def kernel(q, k, attn_mask):
    B, H, S, D = q.shape
    q3 = q.reshape(H, S, D)
    k3 = k.reshape(H, S, D)
    qT = jnp.transpose(q3, (0, 2, 1))
    kT = jnp.transpose(k3, (0, 2, 1))
    m3 = attn_mask.reshape(H, S, S).astype(jnp.int8)

    rq, rk = pl.pallas_call(
        _rank_body,
        grid=(H,),
        in_specs=[
            pl.BlockSpec((1, S, D), lambda h: (h, 0, 0)),
            pl.BlockSpec((1, D, S), lambda h: (h, 0, 0)),
            pl.BlockSpec((1, S, D), lambda h: (h, 0, 0)),
            pl.BlockSpec((1, D, S), lambda h: (h, 0, 0)),
        ],
        out_specs=[
            pl.BlockSpec((1, D, S), lambda h: (h, 0, 0)),
            pl.BlockSpec((1, D, S), lambda h: (h, 0, 0)),
        ],
        out_shape=[
            jax.ShapeDtypeStruct((H, D, S), jnp.int32),
            jax.ShapeDtypeStruct((H, D, S), jnp.int32),
        ],
    )(q3, qT, k3, kT)

    acc = _sc_scatter(qT.reshape(H * D * S), kT.reshape(H * D * S),
                      rq.reshape(H * D * S), rk.reshape(H * D * S))
    acc = acc.reshape(H, S, S)

    out = pl.pallas_call(
        _mask_body,
        grid=(H,),
        in_specs=[
            pl.BlockSpec((1, S, S), lambda h: (h, 0, 0)),
            pl.BlockSpec((1, S, S), lambda h: (h, 0, 0)),
        ],
        out_specs=pl.BlockSpec((1, S, S), lambda h: (h, 0, 0)),
        out_shape=jax.ShapeDtypeStruct((H, S, S), jnp.float32),
    )(acc, m3)
    return out.reshape(B, H, S, S)


# speedup vs baseline: 12.0684x; 1.4766x over previous
import functools

import jax
import jax.numpy as jnp
from jax import lax
from jax.experimental import pallas as pl
from jax.experimental.pallas import tpu as pltpu
from jax.experimental.pallas import tpu_sc as plsc

_H, _S, _D = 8, 512, 64
_NSUB = 16
_CPW = _H * _D // _NSUB
_ROWS = 64
_NPASS = 256 // _ROWS


def _rank_body(q_ref, qT_ref, k_ref, kT_ref, rq_ref, rk_ref):
    S, D = _S, _D
    ii = lax.broadcasted_iota(jnp.int32, (S, S), 0)
    jj = lax.broadcasted_iota(jnp.int32, (S, S), 1)
    i_lt_j = ii < jj

    q = q_ref[0]
    qT = qT_ref[0]
    k = k_ref[0]
    kT = kT_ref[0]
    for d in range(D):
        qc = q[:, d:d + 1]
        qr = qT[d:d + 1, :]
        kc = k[:, d:d + 1]
        kr = kT[d:d + 1, :]
        mq = (qc < qr) | ((qc == qr) & i_lt_j)
        rq_ref[0, d:d + 1, :] = jnp.sum(mq.astype(jnp.int32), axis=0,
                                        keepdims=True)
        mk = (kc < kr) | ((kc == kr) & i_lt_j)
        rk_ref[0, d:d + 1, :] = jnp.sum(mk.astype(jnp.int32), axis=0,
                                        keepdims=True)


def _mask_body(a_ref, m_ref, o_ref):
    o_ref[0] = jnp.where(m_ref[0] != 0, 0.0, a_ref[0] * (1.0 / _D))


_CBLK = 8


def _sc_body(qT_hbm, kT_hbm, rq_hbm, rk_hbm, out_hbm,
             qb, kb, rqb, rkb, qi_a, ki_a, tcolb, twb,
             colb, wb, outbuf, stg_col, stg_w, sem):
    f32 = jnp.float32
    i32 = jnp.int32
    wid = lax.axis_index("s")
    iota = lax.iota(i32, 16)
    nblk = _CPW // _CBLK

    for blk in range(nblk):
        c0 = wid * _CPW + blk * _CBLK
        n = _CBLK * _S
        d1 = pltpu.async_copy(qT_hbm.at[pl.ds(c0 * _S, n)], qb, sem)
        d2 = pltpu.async_copy(kT_hbm.at[pl.ds(c0 * _S, n)], kb, sem)
        d3 = pltpu.async_copy(rq_hbm.at[pl.ds(c0 * _S, n)], rqb, sem)
        d4 = pltpu.async_copy(rk_hbm.at[pl.ds(c0 * _S, n)], rkb, sem)
        d1.wait()
        d2.wait()
        d3.wait()
        d4.wait()

        def per_channel(ch, _):
            base = ch * _S

            def pass_a(t, _):
                sl = pl.ds(base + t * 16, 16)
                iv = iota + t * 16
                plsc.store_scatter(qi_a, [rqb[sl]], iv)
                plsc.store_scatter(ki_a, [rkb[sl]], iv)
                return _
            lax.fori_loop(0, _S // 16, pass_a, None)

            def pass_b(t, _):
                sl = pl.ds(t * 16, 16)
                qiv = qi_a[sl]
                kiv = ki_a[sl]
                qvals = plsc.load_gather(qb, [qiv + base])
                kvals = plsc.load_gather(kb, [kiv + base])
                dqk = qvals - kvals
                w = jnp.exp(-(dqk * dqk))
                plsc.store_scatter(tcolb, [qiv + base], kiv)
                plsc.store_scatter(twb, [qiv + base], w)
                return _
            lax.fori_loop(0, _S // 16, pass_b, None)
            return _
        lax.fori_loop(0, _CBLK, per_channel, None)

        s1 = pltpu.async_copy(tcolb, stg_col.at[pl.ds(c0 * _S, n)], sem)
        s2 = pltpu.async_copy(twb, stg_w.at[pl.ds(c0 * _S, n)], sem)
        s1.wait()
        s2.wait()

    plsc.subcore_barrier()

    head = wid // 2
    zeros16 = jnp.zeros((16,), f32)
    for half in range(_NPASS):
        i0 = (wid % 2) * 256 + half * _ROWS

        descs = []
        for ch in range(_D):
            src = (head * _D + ch) * _S + i0
            descs.append(pltpu.async_copy(
                stg_col.at[pl.ds(src, _ROWS)], colb.at[ch], sem))
            descs.append(pltpu.async_copy(
                stg_w.at[pl.ds(src, _ROWS)], wb.at[ch], sem))

        def zero_chunk(t, _):
            for u in range(8):
                outbuf[pl.ds(t * 128 + u * 16, 16)] = zeros16
            return _
        lax.fori_loop(0, _ROWS * _S // 128, zero_chunk, None)

        for d in descs:
            d.wait()

        def per_ch(ch, _):
            for rb in range(_ROWS // 16):
                sl = pl.ds(rb * 16, 16)
                rows_v = iota + rb * 16
                colv = colb[ch, sl]
                wv = wb[ch, sl]
                plsc.addupdate_scatter(outbuf, [rows_v * _S + colv], wv)
            return _
        lax.fori_loop(0, _D, per_ch, None)

        pltpu.sync_copy(outbuf,
                        out_hbm.at[pl.ds((head * _S + i0) * _S,
                                         _ROWS * _S)])


_sc_scatter = functools.partial(
    pl.kernel,
    out_type=jax.ShapeDtypeStruct((_H * _S * _S,), jnp.float32),
    mesh=plsc.VectorSubcoreMesh(core_axis_name="c", subcore_axis_name="s",
                                num_cores=1),
    compiler_params=pltpu.CompilerParams(needs_layout_passes=False),
    scratch_types=[
        pltpu.VMEM((_CBLK * _S,), jnp.float32),
        pltpu.VMEM((_CBLK * _S,), jnp.float32),
        pltpu.VMEM((_CBLK * _S,), jnp.int32),
        pltpu.VMEM((_CBLK * _S,), jnp.int32),
        pltpu.VMEM((_S,), jnp.int32),
        pltpu.VMEM((_S,), jnp.int32),
        pltpu.VMEM((_CBLK * _S,), jnp.int32),
        pltpu.VMEM((_CBLK * _S,), jnp.float32),
        pltpu.VMEM((_D, _ROWS), jnp.int32),
        pltpu.VMEM((_D, _ROWS), jnp.float32),
        pltpu.VMEM((_ROWS * _S,), jnp.float32),
        pltpu.VMEM_SHARED((_H * _D * _S,), jnp.int32),
        pltpu.VMEM_SHARED((_H * _D * _S,), jnp.float32),
        pltpu.SemaphoreType.DMA,
    ],
)(_sc_body)


def kernel(q, k, attn_mask):
    B, H, S, D = q.shape
    q3 = q.reshape(H, S, D)
    k3 = k.reshape(H, S, D)
    qT = jnp.transpose(q3, (0, 2, 1))
    kT = jnp.transpose(k3, (0, 2, 1))
    m3 = attn_mask.reshape(H, S, S).astype(jnp.int8)

    rq, rk = pl.pallas_call(
        _rank_body,
        grid=(H,),
        in_specs=[
            pl.BlockSpec((1, S, D), lambda h: (h, 0, 0)),
            pl.BlockSpec((1, D, S), lambda h: (h, 0, 0)),
            pl.BlockSpec((1, S, D), lambda h: (h, 0, 0)),
            pl.BlockSpec((1, D, S), lambda h: (h, 0, 0)),
        ],
        out_specs=[
            pl.BlockSpec((1, D, S), lambda h: (h, 0, 0)),
            pl.BlockSpec((1, D, S), lambda h: (h, 0, 0)),
        ],
        out_shape=[
            jax.ShapeDtypeStruct((H, D, S), jnp.int32),
            jax.ShapeDtypeStruct((H, D, S), jnp.int32),
        ],
    )(q3, qT, k3, kT)

    acc = _sc_scatter(qT.reshape(H * D * S), kT.reshape(H * D * S),
                      rq.reshape(H * D * S), rk.reshape(H * D * S))
    acc = acc.reshape(H, S, S)

    out = pl.pallas_call(
        _mask_body,
        grid=(H,),
        in_specs=[
            pl.BlockSpec((1, S, S), lambda h: (h, 0, 0)),
            pl.BlockSpec((1, S, S), lambda h: (h, 0, 0)),
        ],
        out_specs=pl.BlockSpec((1, S, S), lambda h: (h, 0, 0)),
        out_shape=jax.ShapeDtypeStruct((H, S, S), jnp.float32),
    )(acc, m3)
    return out.reshape(B, H, S, S)
